# baseline (speedup 1.0000x reference)
import jax
import jax.numpy as jnp
from jax.experimental import pallas as pl

K = 24


def _main_body(xcs_ref, fcw_ref, fcb_ref, embw_ref, out_ref):
    grp = xcs_ref.shape[0]
    n = xcs_ref.shape[2]

    d2s = []
    for s in range(grp):
        xs = xcs_ref[s]
        et = jax.lax.dot_general(embw_ref[...], xs, (((1,), (0,)), ((), ())),
                                 preferred_element_type=jnp.float32)
        sq = jnp.sum(et * et, axis=0)
        g = jax.lax.dot_general(et, et, (((0,), (0,)), ((), ())),
                                preferred_element_type=jnp.float32)
        d2s.append(jnp.maximum(sq[:, None] + sq[None, :] - 2.0 * g, 0.0))
    d2 = jnp.stack(d2s, axis=1).reshape(n, grp * n)
    bits = jax.lax.bitcast_convert_type(d2, jnp.int32)

    rowi = jax.lax.broadcasted_iota(jnp.int32, (n, grp * n), 0)
    coli = jax.lax.broadcasted_iota(jnp.int32, (n, grp * n), 1)
    eye = (coli % n) == rowi

    ones_row = jnp.ones((1, n), jnp.bfloat16)

    d2od = jnp.where(eye, jnp.inf, d2)
    lo0 = jax.lax.bitcast_convert_type(
        jnp.min(d2od, axis=0, keepdims=True), jnp.int32)
    hi0 = jax.lax.bitcast_convert_type(
        jnp.max(d2, axis=0, keepdims=True), jnp.int32)

    def _step(_, lohi):
        lo, hi = lohi
        mid = lo + (hi - lo) // 2
        p = jnp.where(bits <= mid, 1.0, 0.0).astype(jnp.bfloat16)
        cnt = jax.lax.dot_general(ones_row, p, (((1,), (0,)), ((), ())),
                                  preferred_element_type=jnp.float32)
        ge = cnt >= K
        return jnp.where(ge, lo, mid + 1), jnp.where(ge, mid, hi)

    _, thr = jax.lax.fori_loop(0, 23, _step, (lo0, hi0))

    a = jnp.where((bits <= thr) | eye, 1.0, 0.0).astype(jnp.bfloat16)

    dv = jnp.maximum(
        jax.lax.dot_general(ones_row, a, (((1,), (0,)), ((), ())),
                            preferred_element_type=jnp.float32), 1e-9)

    for s in range(grp):
        xs = xcs_ref[s]
        ag = jax.lax.slice_in_dim(a, s * n, (s + 1) * n, axis=1)
        dvg = jax.lax.slice_in_dim(dv, s * n, (s + 1) * n, axis=1)
        deg = jnp.maximum(
            jax.lax.dot_general(ones_row, ag, (((1,), (1,)), ((), ())),
                                preferred_element_type=jnp.float32), 1e-9)
        xl = jax.lax.dot_general(fcw_ref[...], xs, (((1,), (0,)), ((), ())),
                                 preferred_element_type=jnp.float32)
        xl = xl + fcb_ref[...][:, None]
        yt = jax.lax.dot_general(xl.astype(jnp.bfloat16), ag,
                                 (((1,), (1,)), ((), ())),
                                 preferred_element_type=jnp.float32)
        zt = yt / deg
        hgt = jax.lax.dot_general(zt.astype(jnp.bfloat16), ag,
                                  (((1,), (0,)), ((), ())),
                                  preferred_element_type=jnp.float32)
        out_ref[s] = (hgt / dvg + xs).astype(jnp.bfloat16)


def _bn_silu_body(o_ref, bnw_ref, bnb_ref, y_ref):
    o = o_ref[...].astype(jnp.float32)
    cnt = o.shape[0] * o.shape[2]
    mean = jnp.sum(o, axis=(0, 2), keepdims=True) / cnt
    var = jnp.sum((o - mean) ** 2, axis=(0, 2), keepdims=True) / cnt
    inv = jax.lax.rsqrt(var + 1e-5)
    w = bnw_ref[...][None, :, :]
    b = bnb_ref[...][None, :, :]
    yn = (o - mean) * inv * w + b
    y_ref[...] = yn * (1.0 / (1.0 + jnp.exp(-yn)))


def kernel(x, fc_w, fc_b, emb_w, bn_w, bn_b):
    bb, cc, hh, ww = x.shape
    n = hh * ww
    emb = emb_w.shape[0]
    xcs = x.reshape(bb, cc, n)

    grp = 4
    out = pl.pallas_call(
        _main_body,
        grid=(bb // grp,),
        in_specs=[
            pl.BlockSpec((grp, cc, n), lambda i: (i, 0, 0)),
            pl.BlockSpec((cc, cc), lambda i: (0, 0)),
            pl.BlockSpec((cc,), lambda i: (0,)),
            pl.BlockSpec((emb, cc), lambda i: (0, 0)),
        ],
        out_specs=pl.BlockSpec((grp, cc, n), lambda i: (i, 0, 0)),
        out_shape=jax.ShapeDtypeStruct((bb, cc, n), jnp.bfloat16),
    )(xcs, fc_w, fc_b, emb_w)

    n_cblk = 3
    cb = cc // n_cblk
    y = pl.pallas_call(
        _bn_silu_body,
        grid=(n_cblk,),
        in_specs=[
            pl.BlockSpec((bb, cb, n), lambda i: (0, i, 0)),
            pl.BlockSpec((cb, 1), lambda i: (i, 0)),
            pl.BlockSpec((cb, 1), lambda i: (i, 0)),
        ],
        out_specs=pl.BlockSpec((bb, cb, n), lambda i: (0, i, 0)),
        out_shape=jax.ShapeDtypeStruct((bb, cc, n), jnp.float32),
    )(out, bn_w.reshape(cc, 1), bn_b.reshape(cc, 1))

    return y.reshape(bb, cc, hh, ww)


# speedup vs baseline: 1.1336x; 1.1336x over previous
import jax
import jax.numpy as jnp
from jax.experimental import pallas as pl

K = 24


def _main_body(xcs_ref, fcw_ref, fcb_ref, embw_ref, out_ref):
    xs = xcs_ref[0]
    n = xs.shape[1]

    et = jax.lax.dot_general(embw_ref[...], xs, (((1,), (0,)), ((), ())),
                             preferred_element_type=jnp.float32)
    sq = jnp.sum(et * et, axis=0)
    g = jax.lax.dot_general(et, et, (((0,), (0,)), ((), ())),
                            preferred_element_type=jnp.float32)
    d2 = jnp.maximum(sq[:, None] + sq[None, :] - 2.0 * g, 0.0)
    bits = jax.lax.bitcast_convert_type(d2, jnp.int32)

    rowi = jax.lax.broadcasted_iota(jnp.int32, (n, n), 0)
    coli = jax.lax.broadcasted_iota(jnp.int32, (n, n), 1)
    eye = rowi == coli

    ones_row = jnp.ones((1, n), jnp.bfloat16)
    ones_f32 = jnp.ones((1, n), jnp.float32)

    def _count(p):
        return jax.lax.dot_general(ones_row, p, (((1,), (0,)), ((), ())),
                                   preferred_element_type=jnp.float32)

    d2od = jnp.where(eye, jnp.inf, d2)
    lo0 = jax.lax.bitcast_convert_type(
        jnp.min(d2od, axis=0, keepdims=True), jnp.int32)
    hi0 = jax.lax.bitcast_convert_type(
        jnp.max(d2, axis=0, keepdims=True), jnp.int32)

    def _step(_, lohi):
        lo, hi = lohi
        mid = lo + (hi - lo) // 2
        p = jnp.where(bits <= mid, 1.0, 0.0)
        cnt = jax.lax.dot_general(ones_f32, p, (((1,), (0,)), ((), ())),
                                  preferred_element_type=jnp.float32)
        ge = cnt >= K
        return jnp.where(ge, lo, mid + 1), jnp.where(ge, mid, hi)

    _, thr = jax.lax.fori_loop(0, 23, _step, (lo0, hi0))

    a = jnp.where((bits <= thr) | eye, 1.0, 0.0).astype(jnp.bfloat16)

    dv = jax.lax.dot_general(ones_row, a, (((1,), (0,)), ((), ())),
                             preferred_element_type=jnp.float32)
    de = jax.lax.dot_general(ones_row, a, (((1,), (1,)), ((), ())),
                             preferred_element_type=jnp.float32)
    dv = jnp.maximum(dv, 1e-9)
    de = jnp.maximum(de, 1e-9)

    xl = jax.lax.dot_general(fcw_ref[...], xs, (((1,), (0,)), ((), ())),
                             preferred_element_type=jnp.float32)
    xl = xl + fcb_ref[...][:, None]
    yt = jax.lax.dot_general(xl.astype(jnp.bfloat16), a,
                             (((1,), (1,)), ((), ())),
                             preferred_element_type=jnp.float32)
    zt = yt / de
    hgt = jax.lax.dot_general(zt.astype(jnp.bfloat16), a,
                              (((1,), (0,)), ((), ())),
                              preferred_element_type=jnp.float32)
    out_ref[0] = (hgt / dv + xs).astype(jnp.bfloat16)


def _bn_silu_body(o_ref, bnw_ref, bnb_ref, y_ref):
    o = o_ref[...].astype(jnp.float32)
    cnt = o.shape[0] * o.shape[2]
    mean = jnp.sum(o, axis=(0, 2), keepdims=True) / cnt
    var = jnp.sum((o - mean) ** 2, axis=(0, 2), keepdims=True) / cnt
    inv = jax.lax.rsqrt(var + 1e-5)
    w = bnw_ref[...][None, :, :]
    b = bnb_ref[...][None, :, :]
    yn = (o - mean) * inv * w + b
    y_ref[...] = yn * (1.0 / (1.0 + jnp.exp(-yn)))


def kernel(x, fc_w, fc_b, emb_w, bn_w, bn_b):
    bb, cc, hh, ww = x.shape
    n = hh * ww
    emb = emb_w.shape[0]
    xcs = x.reshape(bb, cc, n)

    out = pl.pallas_call(
        _main_body,
        grid=(bb,),
        in_specs=[
            pl.BlockSpec((1, cc, n), lambda i: (i, 0, 0)),
            pl.BlockSpec((cc, cc), lambda i: (0, 0)),
            pl.BlockSpec((cc,), lambda i: (0,)),
            pl.BlockSpec((emb, cc), lambda i: (0, 0)),
        ],
        out_specs=pl.BlockSpec((1, cc, n), lambda i: (i, 0, 0)),
        out_shape=jax.ShapeDtypeStruct((bb, cc, n), jnp.bfloat16),
    )(xcs, fc_w, fc_b, emb_w)

    n_cblk = 3
    cb = cc // n_cblk
    y = pl.pallas_call(
        _bn_silu_body,
        grid=(n_cblk,),
        in_specs=[
            pl.BlockSpec((bb, cb, n), lambda i: (0, i, 0)),
            pl.BlockSpec((cb, 1), lambda i: (i, 0)),
            pl.BlockSpec((cb, 1), lambda i: (i, 0)),
        ],
        out_specs=pl.BlockSpec((bb, cb, n), lambda i: (0, i, 0)),
        out_shape=jax.ShapeDtypeStruct((bb, cc, n), jnp.float32),
    )(out, bn_w.reshape(cc, 1), bn_b.reshape(cc, 1))

    return y.reshape(bb, cc, hh, ww)


# speedup vs baseline: 1.1337x; 1.0001x over previous
import jax
import jax.numpy as jnp
from jax.experimental import pallas as pl

K = 24


def _main_body(xcs_ref, fcw_ref, fcb_ref, embw_ref, out_ref):
    xs = xcs_ref[0]
    n = xs.shape[1]

    et = jax.lax.dot_general(embw_ref[...], xs, (((1,), (0,)), ((), ())),
                             preferred_element_type=jnp.float32)
    sq = jnp.sum(et * et, axis=0)
    g = jax.lax.dot_general(et, et, (((0,), (0,)), ((), ())),
                            preferred_element_type=jnp.float32)
    d2 = jnp.maximum(sq[:, None] + sq[None, :] - 2.0 * g, 0.0)
    bits = jax.lax.bitcast_convert_type(d2, jnp.int32)

    rowi = jax.lax.broadcasted_iota(jnp.int32, (n, n), 0)
    coli = jax.lax.broadcasted_iota(jnp.int32, (n, n), 1)
    eye = rowi == coli

    ones_row = jnp.ones((1, n), jnp.bfloat16)
    ones_f32 = jnp.ones((1, n), jnp.float32)

    d2od = jnp.where(eye, jnp.inf, d2)
    lo0 = jax.lax.bitcast_convert_type(
        jnp.min(d2od, axis=0, keepdims=True), jnp.int32)
    hi0 = jax.lax.bitcast_convert_type(
        jnp.max(d2, axis=0, keepdims=True), jnp.int32)

    def _step(_, lohi):
        lo, hi = lohi
        mid = lo + (hi - lo) // 2
        p = jnp.where(bits <= mid, 1.0, 0.0)
        cnt = jax.lax.dot_general(ones_f32, p, (((1,), (0,)), ((), ())),
                                  preferred_element_type=jnp.float32)
        ge = cnt >= K
        return jnp.where(ge, lo, mid + 1), jnp.where(ge, mid, hi)

    _, thr = jax.lax.fori_loop(0, 23, _step, (lo0, hi0))

    a = jnp.where((bits <= thr) | eye, 1.0, 0.0).astype(jnp.bfloat16)

    dv = jax.lax.dot_general(ones_row, a, (((1,), (0,)), ((), ())),
                             preferred_element_type=jnp.float32)
    de = jax.lax.dot_general(ones_row, a, (((1,), (1,)), ((), ())),
                             preferred_element_type=jnp.float32)
    dv = jnp.maximum(dv, 1e-9)
    de = jnp.maximum(de, 1e-9)

    xl = jax.lax.dot_general(fcw_ref[...], xs, (((1,), (0,)), ((), ())),
                             preferred_element_type=jnp.float32)
    xl = xl + fcb_ref[...][:, None]
    yt = jax.lax.dot_general(xl.astype(jnp.bfloat16), a,
                             (((1,), (1,)), ((), ())),
                             preferred_element_type=jnp.float32)
    zt = yt / de
    hgt = jax.lax.dot_general(zt.astype(jnp.bfloat16), a,
                              (((1,), (0,)), ((), ())),
                              preferred_element_type=jnp.float32)
    out_ref[0] = (hgt / dv + xs).astype(jnp.bfloat16)


def _bn_silu_body(o_ref, bnw_ref, bnb_ref, y_ref):
    o = o_ref[...].astype(jnp.float32)
    cnt = o.shape[0] * o.shape[2]
    mean = jnp.sum(o, axis=(0, 2), keepdims=True) / cnt
    var = jnp.sum((o - mean) ** 2, axis=(0, 2), keepdims=True) / cnt
    inv = jax.lax.rsqrt(var + 1e-5)
    w = bnw_ref[...][None, :, :]
    b = bnb_ref[...][None, :, :]
    yn = (o - mean) * inv * w + b
    y_ref[...] = yn * (1.0 / (1.0 + jnp.exp(-yn)))


def kernel(x, fc_w, fc_b, emb_w, bn_w, bn_b):
    bb, cc, hh, ww = x.shape
    n = hh * ww
    emb = emb_w.shape[0]
    xcs = x.reshape(bb, cc, n)

    out = pl.pallas_call(
        _main_body,
        grid=(bb,),
        in_specs=[
            pl.BlockSpec((1, cc, n), lambda i: (i, 0, 0)),
            pl.BlockSpec((cc, cc), lambda i: (0, 0)),
            pl.BlockSpec((cc,), lambda i: (0,)),
            pl.BlockSpec((emb, cc), lambda i: (0, 0)),
        ],
        out_specs=pl.BlockSpec((1, cc, n), lambda i: (i, 0, 0)),
        out_shape=jax.ShapeDtypeStruct((bb, cc, n), jnp.bfloat16),
    )(xcs, fc_w, fc_b, emb_w)

    n_cblk = 3
    cb = cc // n_cblk
    y = pl.pallas_call(
        _bn_silu_body,
        grid=(n_cblk,),
        in_specs=[
            pl.BlockSpec((bb, cb, n), lambda i: (0, i, 0)),
            pl.BlockSpec((cb, 1), lambda i: (i, 0)),
            pl.BlockSpec((cb, 1), lambda i: (i, 0)),
        ],
        out_specs=pl.BlockSpec((bb, cb, n), lambda i: (0, i, 0)),
        out_shape=jax.ShapeDtypeStruct((bb, cc, n), jnp.float32),
    )(out, bn_w.reshape(cc, 1), bn_b.reshape(cc, 1))

    return y.reshape(bb, cc, hh, ww)
